# baseline (speedup 1.0000x reference)
import functools

import jax
import jax.numpy as jnp
from jax.experimental import pallas as pl
from jax.experimental.pallas import tpu as pltpu

N = 8192
BI = 256
BJ = 2048
NEG = float("-inf")


def _body(row_ref, col_ref, adj_ref, out_ref, acc_ref, *, nj):
    j = pl.program_id(1)

    t = jnp.dot(row_ref[:, :], col_ref[:, :],
                preferred_element_type=jnp.float32)
    adj = adj_ref[:, :]

    part = None
    for c in range(BJ // 128):
        sl = slice(c * 128, (c + 1) * 128)
        piece = jnp.where(adj[:, sl] != 0, t[:, sl], NEG)
        part = piece if part is None else jnp.maximum(part, piece)

    @pl.when(j == 0)
    def _():
        acc_ref[:, :] = part

    @pl.when(j > 0)
    def _():
        acc_ref[:, :] = jnp.maximum(acc_ref[:, :], part)

    @pl.when(j == nj - 1)
    def _():
        acc = jnp.max(acc_ref[:, :], axis=1, keepdims=True)
        d2 = acc + row_ref[:, 6:7]
        maxd = jnp.sqrt(jnp.maximum(d2, 0.0))
        prev = row_ref[:, 4:5]
        phimean = row_ref[:, 5:6]
        out_ref[:, :] = 0.5 * (prev + maxd * phimean)


@jax.jit
def kernel(previous_inclusion_score, nodes, adjacency_matrix, W_phi, W_theta):
    w = W_theta[:, 0]
    wx = nodes * w[None, :]
    sq = jnp.sum(wx * wx, axis=1)
    phimean = jnp.mean(W_phi)

    zeros = jnp.zeros((N,), jnp.float32)
    ones = jnp.ones((N,), jnp.float32)
    rows = jnp.stack(
        [wx[:, 0], wx[:, 1], wx[:, 2], ones,
         previous_inclusion_score, jnp.full((N,), phimean),
         sq, zeros], axis=1)
    cols = jnp.stack(
        [-2.0 * wx[:, 0], -2.0 * wx[:, 1], -2.0 * wx[:, 2], sq,
         zeros, zeros, zeros, zeros], axis=0)

    ni = N // BI
    nj = N // BJ
    out = pl.pallas_call(
        functools.partial(_body, nj=nj),
        grid=(ni, nj),
        in_specs=[
            pl.BlockSpec((BI, 8), lambda i, j: (i, 0)),
            pl.BlockSpec((8, BJ), lambda i, j: (0, j)),
            pl.BlockSpec((BI, BJ), lambda i, j: (i, j)),
        ],
        out_specs=pl.BlockSpec((BI, 1), lambda i, j: (i, 0)),
        out_shape=jax.ShapeDtypeStruct((N, 1), jnp.float32),
        scratch_shapes=[pltpu.VMEM((BI, 128), jnp.float32)],
        compiler_params=pltpu.CompilerParams(
            dimension_semantics=("parallel", "arbitrary")),
    )(rows, cols, adjacency_matrix)
    return out[:, 0]


# speedup vs baseline: 1.6126x; 1.6126x over previous
import jax
import jax.numpy as jnp
from jax.experimental import pallas as pl
from jax.experimental.pallas import tpu as pltpu

N = 8192
BI = 256
CH = 2048
NEG = float("-inf")


def _body(row_ref, col_ref, adj_ref, out_ref):
    part = None
    for c in range(N // CH):
        sl = slice(c * CH, (c + 1) * CH)
        t = jnp.dot(row_ref[:, :], col_ref[:, sl],
                    preferred_element_type=jnp.float32)
        adj = adj_ref[:, sl]
        for s in range(CH // 128):
            ssl = slice(s * 128, (s + 1) * 128)
            piece = jnp.where(adj[:, ssl] != 0, t[:, ssl], NEG)
            part = piece if part is None else jnp.maximum(part, piece)

    acc = jnp.max(part, axis=1, keepdims=True)
    d2 = acc + row_ref[:, 6:7]
    maxd = jnp.sqrt(jnp.maximum(d2, 0.0))
    prev = row_ref[:, 4:5]
    phimean = row_ref[:, 5:6]
    out_ref[:, :] = 0.5 * (prev + maxd * phimean)


@jax.jit
def kernel(previous_inclusion_score, nodes, adjacency_matrix, W_phi, W_theta):
    w = W_theta[:, 0]
    wx = nodes * w[None, :]
    sq = jnp.sum(wx * wx, axis=1)
    phimean = jnp.mean(W_phi)

    zeros = jnp.zeros((N,), jnp.float32)
    ones = jnp.ones((N,), jnp.float32)
    rows = jnp.stack(
        [wx[:, 0], wx[:, 1], wx[:, 2], ones,
         previous_inclusion_score, jnp.full((N,), phimean),
         sq, zeros], axis=1)
    cols = jnp.stack(
        [-2.0 * wx[:, 0], -2.0 * wx[:, 1], -2.0 * wx[:, 2], sq,
         zeros, zeros, zeros, zeros], axis=0)

    ni = N // BI
    out = pl.pallas_call(
        _body,
        grid=(ni,),
        in_specs=[
            pl.BlockSpec((BI, 8), lambda i: (i, 0)),
            pl.BlockSpec((8, N), lambda i: (0, 0)),
            pl.BlockSpec((BI, N), lambda i: (i, 0)),
        ],
        out_specs=pl.BlockSpec((BI, 1), lambda i: (i, 0)),
        out_shape=jax.ShapeDtypeStruct((N, 1), jnp.float32),
        compiler_params=pltpu.CompilerParams(
            dimension_semantics=("arbitrary",)),
    )(rows, cols, adjacency_matrix)
    return out[:, 0]


# speedup vs baseline: 1.6151x; 1.0016x over previous
import jax
import jax.numpy as jnp
from jax.experimental import pallas as pl
from jax.experimental.pallas import tpu as pltpu

N = 8192
BI = 512
CH = 2048
NEG = float("-inf")


def _body(row_ref, col_ref, adj_ref, out_ref):
    part = None
    for c in range(N // CH):
        sl = slice(c * CH, (c + 1) * CH)
        t = jnp.dot(row_ref[:, :], col_ref[:, sl],
                    preferred_element_type=jnp.float32)
        adj = adj_ref[:, sl]
        for s in range(CH // 128):
            ssl = slice(s * 128, (s + 1) * 128)
            piece = jnp.where(adj[:, ssl] != 0, t[:, ssl], NEG)
            part = piece if part is None else jnp.maximum(part, piece)

    acc = jnp.max(part, axis=1, keepdims=True)
    d2 = acc + row_ref[:, 6:7]
    maxd = jnp.sqrt(jnp.maximum(d2, 0.0))
    prev = row_ref[:, 4:5]
    phimean = row_ref[:, 5:6]
    out_ref[:, :] = 0.5 * (prev + maxd * phimean)


@jax.jit
def kernel(previous_inclusion_score, nodes, adjacency_matrix, W_phi, W_theta):
    w = W_theta[:, 0]
    wx = nodes * w[None, :]
    sq = jnp.sum(wx * wx, axis=1)
    phimean = jnp.mean(W_phi)

    zeros = jnp.zeros((N,), jnp.float32)
    ones = jnp.ones((N,), jnp.float32)
    rows = jnp.stack(
        [wx[:, 0], wx[:, 1], wx[:, 2], ones,
         previous_inclusion_score, jnp.full((N,), phimean),
         sq, zeros], axis=1)
    cols = jnp.stack(
        [-2.0 * wx[:, 0], -2.0 * wx[:, 1], -2.0 * wx[:, 2], sq,
         zeros, zeros, zeros, zeros], axis=0)

    ni = N // BI
    out = pl.pallas_call(
        _body,
        grid=(ni,),
        in_specs=[
            pl.BlockSpec((BI, 8), lambda i: (i, 0)),
            pl.BlockSpec((8, N), lambda i: (0, 0)),
            pl.BlockSpec((BI, N), lambda i: (i, 0)),
        ],
        out_specs=pl.BlockSpec((BI, 1), lambda i: (i, 0)),
        out_shape=jax.ShapeDtypeStruct((N, 1), jnp.float32),
        compiler_params=pltpu.CompilerParams(
            dimension_semantics=("arbitrary",)),
    )(rows, cols, adjacency_matrix)
    return out[:, 0]
